# baseline (speedup 1.0000x reference)
import functools

import jax
import jax.numpy as jnp
from jax import lax
from jax.experimental import pallas as pl
from jax.experimental.pallas import tpu as pltpu
from jax.experimental.pallas import tpu_sc as plsc

N = 10000
DIN = 128
H = 64
HA = 80
NC, NS, NL = 2, 16, 16
NW = NC * NS
CHUNK = 128
N_PAD = 10112
RPT = N_PAD // NS
BM = 1000


NBUF = 4
KA = 64
KB = 96


def _sc_agg_body(d, ka, kb, feat, src, dst, parts,
                 r0b, r1b, r2b, r3b,
                 is0, is1, is2, is3, idx_db, zbuf, acc,
                 sem0, sem1, sem2, sem3):
    rows = (r0b, r1b, r2b, r3b)
    isb = (is0, is1, is2, is3)
    sems = (sem0, sem1, sem2, sem3)
    cid = lax.axis_index("c")
    sid = lax.axis_index("s")
    wid = sid * NC + cid
    zv = jnp.zeros((NL,), jnp.float32)

    def zrow(i, c):
        for j in range(d // NL):
            zbuf[i, pl.ds(j * NL, NL)] = zv
        return c
    lax.fori_loop(0, CHUNK, zrow, 0)
    r0 = sid * RPT
    off = 0
    while off < RPT:
        sz = min(CHUNK, RPT - off)
        pltpu.sync_copy(zbuf.at[pl.ds(0, sz)], acc.at[pl.ds(r0 + off, sz)])
        off += sz

    pair = ka + kb
    base0 = (sid * pair + cid * ka) * CHUNK
    my_chunks = jnp.where(cid == 0, ka, kb)
    n_groups = my_chunks // NBUF

    def _load_idx(arr, c, buf):
        base = pl.multiple_of(base0 + c * CHUNK, CHUNK)
        pltpu.sync_copy(arr.at[pl.ds(base, CHUNK)], buf)

    for b in range(NBUF):
        _load_idx(src, b, isb[b])
        pltpu.async_copy(feat.at[isb[b]], rows[b], sems[b])
    plsc.subcore_barrier()

    def group_body(g, carry):
        for b in range(NBUF):
            c = g * NBUF + b
            pltpu.make_async_copy(feat.at[isb[b]], rows[b], sems[b]).wait()
            _load_idx(dst, c, idx_db)
            pltpu.sync_copy(rows[b], acc.at[idx_db], add=True)

            @pl.when(g < n_groups - 1)
            def _():
                _load_idx(src, c + NBUF, isb[b])
                pltpu.async_copy(feat.at[isb[b]], rows[b], sems[b])
        return carry

    lax.fori_loop(0, n_groups, group_body, 0)
    plsc.subcore_barrier()
    pltpu.sync_copy(acc.at[pl.ds(r0, RPT)], parts.at[cid, pl.ds(r0, RPT)])


def _make_sc_agg(d, ka, kb):
    body = functools.partial(_sc_agg_body, d, ka, kb)
    return pl.kernel(
        body,
        out_type=jax.ShapeDtypeStruct((NC, N_PAD, d), jnp.float32),
        mesh=plsc.VectorSubcoreMesh(core_axis_name="c", subcore_axis_name="s"),
        scratch_types=[
            pltpu.VMEM((CHUNK, d), jnp.float32),
            pltpu.VMEM((CHUNK, d), jnp.float32),
            pltpu.VMEM((CHUNK, d), jnp.float32),
            pltpu.VMEM((CHUNK, d), jnp.float32),
            pltpu.VMEM((CHUNK,), jnp.int32),
            pltpu.VMEM((CHUNK,), jnp.int32),
            pltpu.VMEM((CHUNK,), jnp.int32),
            pltpu.VMEM((CHUNK,), jnp.int32),
            pltpu.VMEM((CHUNK,), jnp.int32),
            pltpu.VMEM((CHUNK, d), jnp.float32),
            pltpu.VMEM_SHARED((N_PAD, d), jnp.float32),
            pltpu.SemaphoreType.DMA,
            pltpu.SemaphoreType.DMA,
            pltpu.SemaphoreType.DMA,
            pltpu.SemaphoreType.DMA,
        ],
        compiler_params=pltpu.CompilerParams(use_tc_tiling_on_sc=False),
    )


def _enc_mm(x_ref, w_ref, o_ref):
    mm = jnp.dot(x_ref[...], w_ref[...], preferred_element_type=jnp.float32)
    col = lax.broadcasted_iota(jnp.int32, (BM, HA), 1)
    o_ref[...] = mm + jnp.where(col == H, 1.0, 0.0)


def _norm_tanh(p_ref, b_ref, z_ref, deg_ref):
    p = p_ref[0] + p_ref[1]
    deg = jnp.maximum(p[:, H:H + 1], 1.0)
    z_ref[...] = jnp.tanh(p[:, :H] / deg + b_ref[...])
    deg_ref[...] = deg


def _dec_mm(q_ref, deg_ref, w_ref, db_ref, o_ref):
    q = (q_ref[0] + q_ref[1]) / deg_ref[...]
    o_ref[...] = jnp.dot(q, w_ref[...],
                         preferred_element_type=jnp.float32) + db_ref[...]


def kernel(x, edge_index, y, W_enc, b_enc, W_dec, dom_bias):
    e = edge_index.shape[1]
    e_pad = NS * (KA + KB) * CHUNK
    assert e_pad >= e
    src = edge_index[0].astype(jnp.int32)
    dst = edge_index[1].astype(jnp.int32)
    if e_pad != e:
        src = jnp.concatenate([src, jnp.zeros((e_pad - e,), jnp.int32)])
        fill = N + jnp.arange(e_pad - e, dtype=jnp.int32) % (N_PAD - N)
        dst = jnp.concatenate([dst, fill])
    w_pad = jnp.concatenate(
        [W_enc, jnp.zeros((DIN, HA - H), jnp.float32)], axis=1)

    h = pl.pallas_call(
        _enc_mm,
        grid=(N // BM,),
        in_specs=[pl.BlockSpec((BM, DIN), lambda i: (i, 0)),
                  pl.BlockSpec((DIN, HA), lambda i: (0, 0))],
        out_specs=pl.BlockSpec((BM, HA), lambda i: (i, 0)),
        out_shape=jax.ShapeDtypeStruct((N, HA), jnp.float32),
    )(x, w_pad)

    parts = _make_sc_agg(HA, KA, KB)(h, src, dst)

    z, deg = pl.pallas_call(
        _norm_tanh,
        grid=(N // BM,),
        in_specs=[pl.BlockSpec((2, BM, HA), lambda i: (0, i, 0)),
                  pl.BlockSpec((1, H), lambda i: (0, 0))],
        out_specs=[pl.BlockSpec((BM, H), lambda i: (i, 0)),
                   pl.BlockSpec((BM, 1), lambda i: (i, 0))],
        out_shape=[jax.ShapeDtypeStruct((N, H), jnp.float32),
                   jax.ShapeDtypeStruct((N, 1), jnp.float32)],
    )(parts, b_enc.reshape(1, H))

    qparts = _make_sc_agg(H, KA, KB)(z, src, dst)

    recon = pl.pallas_call(
        _dec_mm,
        grid=(N // BM,),
        in_specs=[pl.BlockSpec((2, BM, H), lambda i: (0, i, 0)),
                  pl.BlockSpec((BM, 1), lambda i: (i, 0)),
                  pl.BlockSpec((H, DIN), lambda i: (0, 0)),
                  pl.BlockSpec((1, DIN), lambda i: (0, 0))],
        out_specs=pl.BlockSpec((BM, DIN), lambda i: (i, 0)),
        out_shape=jax.ShapeDtypeStruct((N, DIN), jnp.float32),
    )(qparts, deg, W_dec, dom_bias)

    return recon


# speedup vs baseline: 1.9766x; 1.9766x over previous
import functools

import jax
import jax.numpy as jnp
from jax import lax
from jax.experimental import pallas as pl
from jax.experimental.pallas import tpu as pltpu
from jax.experimental.pallas import tpu_sc as plsc

N = 10000
DIN = 128
H = 64
HA = 80
NC, NS, NL = 2, 16, 16
NW = NC * NS
CHUNK = 128
N_PAD = 10112
RPT = N_PAD // NS
BM = 1000


NBUF = 4
KA = 40
KB = 39


def _sc_agg_body(d, ka, kb, feat, src, dst, parts,
                 r0b, r1b, r2b, r3b,
                 is0, is1, is2, is3, idx_db, zbuf, acc,
                 sem0, sem1, sem2, sem3):
    rows = (r0b, r1b, r2b, r3b)
    isb = (is0, is1, is2, is3)
    sems = (sem0, sem1, sem2, sem3)
    cid = lax.axis_index("c")
    sid = lax.axis_index("s")
    wid = sid * NC + cid
    zv = jnp.zeros((NL,), jnp.float32)

    def zrow(i, c):
        for j in range(d // NL):
            zbuf[i, pl.ds(j * NL, NL)] = zv
        return c
    lax.fori_loop(0, CHUNK, zrow, 0)
    r0 = sid * RPT
    off = 0
    while off < RPT:
        sz = min(CHUNK, RPT - off)
        pltpu.sync_copy(zbuf.at[pl.ds(0, sz)], acc.at[pl.ds(r0 + off, sz)])
        off += sz

    n_chunks = ka + kb
    per_w = n_chunks * CHUNK
    base0 = wid * per_w

    def _load_idx(arr, c, buf):
        base = pl.multiple_of(base0 + c * CHUNK, CHUNK)
        pltpu.sync_copy(arr.at[pl.ds(base, CHUNK)], buf)

    descs = [None] * n_chunks
    for b in range(NBUF):
        _load_idx(src, b, isb[b])
        descs[b] = pltpu.async_copy(feat.at[isb[b]], rows[b], sems[b])
    plsc.subcore_barrier()

    for c in range(n_chunks):
        b = c % NBUF
        descs[c].wait()
        _load_idx(dst, c, idx_db)
        pltpu.sync_copy(rows[b], acc.at[idx_db], add=True)
        if c + NBUF < n_chunks:
            _load_idx(src, c + NBUF, isb[b])
            descs[c + NBUF] = pltpu.async_copy(
                feat.at[isb[b]], rows[b], sems[b])
    plsc.subcore_barrier()
    pltpu.sync_copy(acc.at[pl.ds(r0, RPT)], parts.at[cid, pl.ds(r0, RPT)])


def _make_sc_agg(d, ka, kb):
    body = functools.partial(_sc_agg_body, d, ka, kb)
    return pl.kernel(
        body,
        out_type=jax.ShapeDtypeStruct((NC, N_PAD, d), jnp.float32),
        mesh=plsc.VectorSubcoreMesh(core_axis_name="c", subcore_axis_name="s"),
        scratch_types=[
            pltpu.VMEM((CHUNK, d), jnp.float32),
            pltpu.VMEM((CHUNK, d), jnp.float32),
            pltpu.VMEM((CHUNK, d), jnp.float32),
            pltpu.VMEM((CHUNK, d), jnp.float32),
            pltpu.VMEM((CHUNK,), jnp.int32),
            pltpu.VMEM((CHUNK,), jnp.int32),
            pltpu.VMEM((CHUNK,), jnp.int32),
            pltpu.VMEM((CHUNK,), jnp.int32),
            pltpu.VMEM((CHUNK,), jnp.int32),
            pltpu.VMEM((CHUNK, d), jnp.float32),
            pltpu.VMEM_SHARED((N_PAD, d), jnp.float32),
            pltpu.SemaphoreType.DMA,
            pltpu.SemaphoreType.DMA,
            pltpu.SemaphoreType.DMA,
            pltpu.SemaphoreType.DMA,
        ],
        compiler_params=pltpu.CompilerParams(use_tc_tiling_on_sc=False),
    )


def _enc_mm(x_ref, w_ref, o_ref):
    mm = jnp.dot(x_ref[...], w_ref[...], preferred_element_type=jnp.float32)
    col = lax.broadcasted_iota(jnp.int32, (BM, HA), 1)
    o_ref[...] = mm + jnp.where(col == H, 1.0, 0.0)


def _norm_tanh(p_ref, b_ref, z_ref, deg_ref):
    p = p_ref[0] + p_ref[1]
    deg = jnp.maximum(p[:, H:H + 1], 1.0)
    z_ref[...] = jnp.tanh(p[:, :H] / deg + b_ref[...])
    deg_ref[...] = deg


def _dec_mm(q_ref, deg_ref, w_ref, db_ref, o_ref):
    q = (q_ref[0] + q_ref[1]) / deg_ref[...]
    o_ref[...] = jnp.dot(q, w_ref[...],
                         preferred_element_type=jnp.float32) + db_ref[...]


def kernel(x, edge_index, y, W_enc, b_enc, W_dec, dom_bias):
    e = edge_index.shape[1]
    e_pad = NW * (KA + KB) * CHUNK
    assert e_pad >= e
    src = edge_index[0].astype(jnp.int32)
    dst = edge_index[1].astype(jnp.int32)
    if e_pad != e:
        pad_n = e_pad - e
        ar = jnp.arange(pad_n, dtype=jnp.int32)
        src = jnp.concatenate([src, (ar * 79) % N])
        dst = jnp.concatenate([dst, N + ar % (N_PAD - N)])
    w_pad = jnp.concatenate(
        [W_enc, jnp.zeros((DIN, HA - H), jnp.float32)], axis=1)

    h = pl.pallas_call(
        _enc_mm,
        grid=(N // BM,),
        in_specs=[pl.BlockSpec((BM, DIN), lambda i: (i, 0)),
                  pl.BlockSpec((DIN, HA), lambda i: (0, 0))],
        out_specs=pl.BlockSpec((BM, HA), lambda i: (i, 0)),
        out_shape=jax.ShapeDtypeStruct((N, HA), jnp.float32),
    )(x, w_pad)

    parts = _make_sc_agg(HA, KA, KB)(h, src, dst)

    z, deg = pl.pallas_call(
        _norm_tanh,
        grid=(N // BM,),
        in_specs=[pl.BlockSpec((2, BM, HA), lambda i: (0, i, 0)),
                  pl.BlockSpec((1, H), lambda i: (0, 0))],
        out_specs=[pl.BlockSpec((BM, H), lambda i: (i, 0)),
                   pl.BlockSpec((BM, 1), lambda i: (i, 0))],
        out_shape=[jax.ShapeDtypeStruct((N, H), jnp.float32),
                   jax.ShapeDtypeStruct((N, 1), jnp.float32)],
    )(parts, b_enc.reshape(1, H))

    qparts = _make_sc_agg(H, KA, KB)(z, src, dst)

    recon = pl.pallas_call(
        _dec_mm,
        grid=(N // BM,),
        in_specs=[pl.BlockSpec((2, BM, H), lambda i: (0, i, 0)),
                  pl.BlockSpec((BM, 1), lambda i: (i, 0)),
                  pl.BlockSpec((H, DIN), lambda i: (0, 0)),
                  pl.BlockSpec((1, DIN), lambda i: (0, 0))],
        out_specs=pl.BlockSpec((BM, DIN), lambda i: (i, 0)),
        out_shape=jax.ShapeDtypeStruct((N, DIN), jnp.float32),
    )(qparts, deg, W_dec, dom_bias)

    return recon
